# baseline (speedup 1.0000x reference)
import jax
import jax.numpy as jnp
from jax.experimental import pallas as pl
from jax.experimental.pallas import tpu as pltpu

_D = 64
_K = 1024
_SS = 8


def _vq_kernel(z_ref, w_ref, out_ref):
    w = w_ref[...]
    ones = jnp.ones((z_ref.shape[0], 128), jnp.bfloat16)
    cols = []
    for si in range(_SS):
        x = z_ref[:, 128 * si:128 * (si + 1)].astype(jnp.bfloat16)
        x2 = jnp.concatenate([x, ones], axis=1)
        t = jnp.dot(x2, w, preferred_element_type=jnp.float32,
                    precision=jax.lax.Precision.DEFAULT)
        te = t[:, :_K]
        to = t[:, _K:]
        ie = jnp.argmax(te, axis=1).astype(jnp.float32)[:, None]
        io = jnp.argmax(to, axis=1).astype(jnp.float32)[:, None]
        cols.append(ie)
        cols.append(io)
    tile = jnp.concatenate(cols, axis=1)
    out_ref[...] = tile.T.astype(jnp.int32)


def kernel(z, codebook):
    B, L = z.shape
    P = L // _D
    cbt = codebook.T.astype(jnp.bfloat16)
    zcb = jnp.zeros_like(cbt)
    w2 = jnp.concatenate(
        [jnp.concatenate([cbt, zcb], 1), jnp.concatenate([zcb, cbt], 1)], 0)
    hcsq = 0.5 * jnp.sum(codebook * codebook, axis=1)
    hcsq2 = jnp.concatenate([hcsq, hcsq])[None, :]
    a0 = hcsq2.astype(jnp.bfloat16)
    r1 = hcsq2 - a0.astype(jnp.float32)
    a1 = r1.astype(jnp.bfloat16)
    a2 = (r1 - a1.astype(jnp.float32)).astype(jnp.bfloat16)
    w4 = jnp.concatenate(
        [w2, -a0, -a1, -a2, jnp.zeros((125, 2 * _K), jnp.bfloat16)], 0)

    grid = (L // (128 * _SS),)
    out = pl.pallas_call(
        _vq_kernel,
        grid=grid,
        in_specs=[
            pl.BlockSpec((B, 128 * _SS), lambda g: (0, g)),
            pl.BlockSpec((4 * _D, 2 * _K), lambda g: (0, 0)),
        ],
        out_specs=pl.BlockSpec((2 * _SS, B), lambda g: (g, 0)),
        out_shape=jax.ShapeDtypeStruct((P, B), jnp.int32),
    )(z, w4)
    return out.T


# speedup vs baseline: 1.1320x; 1.1320x over previous
import jax
import jax.numpy as jnp
from jax.experimental import pallas as pl
from jax.experimental.pallas import tpu as pltpu

_D = 64
_K = 1024
_SS = 16


def _vq_kernel(z_ref, w_ref, hcsq_ref, out_ref):
    w = w_ref[...]
    hcsq = hcsq_ref[...]
    cols = []
    for si in range(_SS):
        x = z_ref[:, 128 * si:128 * (si + 1)].astype(jnp.bfloat16)
        sc = jnp.dot(x, w, preferred_element_type=jnp.float32,
                     precision=jax.lax.Precision.DEFAULT)
        t = sc - hcsq
        te = t[:, :_K]
        to = t[:, _K:]
        ie = jnp.argmax(te, axis=1).astype(jnp.float32)[:, None]
        io = jnp.argmax(to, axis=1).astype(jnp.float32)[:, None]
        cols.append(ie)
        cols.append(io)
    tile = jnp.concatenate(cols, axis=1)
    out_ref[...] = tile.T.astype(jnp.int32)


def kernel(z, codebook):
    B, L = z.shape
    P = L // _D
    cbt = codebook.T.astype(jnp.bfloat16)
    zcb = jnp.zeros_like(cbt)
    w2 = jnp.concatenate(
        [jnp.concatenate([cbt, zcb], 1), jnp.concatenate([zcb, cbt], 1)], 0)
    hcsq = 0.5 * jnp.sum(codebook * codebook, axis=1)
    hcsq2 = jnp.concatenate([hcsq, hcsq])[None, :]

    grid = (L // (128 * _SS),)
    out = pl.pallas_call(
        _vq_kernel,
        grid=grid,
        in_specs=[
            pl.BlockSpec((B, 128 * _SS), lambda g: (0, g)),
            pl.BlockSpec((2 * _D, 2 * _K), lambda g: (0, 0)),
            pl.BlockSpec((1, 2 * _K), lambda g: (0, 0)),
        ],
        out_specs=pl.BlockSpec((2 * _SS, B), lambda g: (g, 0)),
        out_shape=jax.ShapeDtypeStruct((P, B), jnp.int32),
    )(z, w2, hcsq2)
    return out.T


# speedup vs baseline: 1.1327x; 1.0007x over previous
import jax
import jax.numpy as jnp
from jax.experimental import pallas as pl

_D = 64
_K = 1024
_SS = 16


def _vq_kernel(z_ref, w_ref, hcsq_ref, out_ref):
    w = w_ref[...]
    hcsq = hcsq_ref[...]
    cols = []
    for si in range(_SS):
        x = z_ref[:, 128 * si:128 * (si + 1)].astype(jnp.bfloat16)
        sc = jnp.dot(x, w, preferred_element_type=jnp.float32,
                     precision=jax.lax.Precision.DEFAULT)
        t = sc - hcsq
        te = t[:, :_K]
        to = t[:, _K:]
        ie = jnp.argmax(te, axis=1).astype(jnp.float32)[:, None]
        io = jnp.argmax(to, axis=1).astype(jnp.float32)[:, None]
        cols.append(ie)
        cols.append(io)
    tile = jnp.concatenate(cols, axis=1)
    out_ref[...] = tile.T.astype(jnp.int32)


def kernel(z, codebook):
    B, L = z.shape
    P = L // _D
    cbt = codebook.T.astype(jnp.bfloat16)
    zcb = jnp.zeros_like(cbt)
    w2 = jnp.concatenate(
        [jnp.concatenate([cbt, zcb], 1), jnp.concatenate([zcb, cbt], 1)], 0)
    hcsq = 0.5 * jnp.sum(codebook * codebook, axis=1)
    hcsq2 = jnp.concatenate([hcsq, hcsq])[None, :]

    grid = (L // (128 * _SS),)
    out = pl.pallas_call(
        _vq_kernel,
        grid=grid,
        in_specs=[
            pl.BlockSpec((B, 128 * _SS), lambda g: (0, g)),
            pl.BlockSpec((2 * _D, 2 * _K), lambda g: (0, 0)),
            pl.BlockSpec((1, 2 * _K), lambda g: (0, 0)),
        ],
        out_specs=pl.BlockSpec((2 * _SS, B), lambda g: (g, 0)),
        out_shape=jax.ShapeDtypeStruct((P, B), jnp.int32),
    )(z, w2, hcsq2)
    return out.T
